# baseline (speedup 1.0000x reference)
import functools

import jax
import jax.numpy as jnp
from jax import lax
from jax.experimental import pallas as pl
from jax.experimental.pallas import tpu as pltpu
from jax.experimental.pallas import tpu_sc as plsc

N_IN = 4096
N_OUT = 16384
KTOP = max(1, int(0.01 * N_OUT))
TAU = 0.1
BC = 1024
NB = N_OUT // BC

NW = 16
CHUNK = N_OUT // NW
NG = CHUNK // 16

FINE_PAD = 1280
COARSE_PAD = 128


def _matvec_body(a_ref, w_ref, y_ref):
    x = jnp.maximum(
        jnp.dot(a_ref[...], w_ref[...], preferred_element_type=jnp.float32), 0.0)
    y_ref[...] = x / (x + 1.0)


def _matvec(a_ECin, W):
    return pl.pallas_call(
        _matvec_body,
        grid=(NB,),
        in_specs=[
            pl.BlockSpec((1, N_IN), lambda i: (0, 0)),
            pl.BlockSpec((N_IN, BC), lambda i: (0, i)),
        ],
        out_specs=pl.BlockSpec((1, BC), lambda i: (0, i)),
        out_shape=jax.ShapeDtypeStruct((1, N_OUT), jnp.float32),
        compiler_params=pltpu.CompilerParams(
            dimension_semantics=("arbitrary",)),
    )(a_ECin.reshape(1, N_IN), W).reshape(N_OUT)


def _iota16():
    return lax.iota(jnp.int32, 16)


def _suffix(v):
    return lax.rev(jnp.cumsum(lax.rev(v, (0,))), (0,))


def _pick(vec, idx):
    return jnp.sum(jnp.where(_iota16() == idx, vec, 0))


def _sc_body(y_hbm, act_hbm, out_hbm, yv, actv, outv, bitsv, idxf, idxc,
             ones_v, zbuf, histc_l, chunk_l, sem,
             histf0, histf1, histf2, histc0, histc1, histc2):
    wid = lax.axis_index("s")
    base = wid * CHUNK

    pltpu.sync_copy(y_hbm.at[pl.ds(base, CHUNK)], yv)
    pltpu.sync_copy(act_hbm.at[pl.ds(base, CHUNK)], actv)
    for j in range(8):
        ones_v[pl.ds(j * 16, 16)] = jnp.ones((16,), jnp.int32)
    for j in range(5):
        zbuf[pl.ds(j * 16, 16)] = jnp.zeros((16,), jnp.int32)
    for g in range(NG):
        bitsv[pl.ds(g * 16, 16)] = plsc.bitcast(yv[pl.ds(g * 16, 16)],
                                                jnp.int32)

    for h in (histf0, histf1, histf2):
        pltpu.sync_copy(zbuf.at[pl.ds(0, FINE_PAD // NW)],
                        h.at[pl.ds(wid * (FINE_PAD // NW), FINE_PAD // NW)])
    for h in (histc0, histc1, histc2):
        pltpu.sync_copy(zbuf.at[pl.ds(0, COARSE_PAD // NW)],
                        h.at[pl.ds(wid * (COARSE_PAD // NW), COARSE_PAD // NW)])
    plsc.subcore_barrier()

    prefix = jnp.int32(0)
    rem = jnp.int32(KTOP)
    for r, sh in enumerate((20, 10, 0)):
        histf = (histf0, histf1, histf2)[r]
        histc = (histc0, histc1, histc2)[r]

        for g in range(NG):
            b = bitsv[pl.ds(g * 16, 16)]
            binf = (b >> sh) & 1023
            if r > 0:
                binf = jnp.where((b >> (sh + 10)) == prefix, binf, 1024)
            idxf[g // 8, pl.ds((g % 8) * 16, 16)] = binf
            idxc[g // 8, pl.ds((g % 8) * 16, 16)] = binf >> 4

        copies = [pltpu.async_copy(ones_v, histf.at[idxf.at[j]], sem, add=True)
                  for j in range(8)]
        copies += [pltpu.async_copy(ones_v, histc.at[idxc.at[j]], sem, add=True)
                   for j in range(8)]
        for c in copies:
            c.wait()
        plsc.subcore_barrier()

        pltpu.sync_copy(histc.at[pl.ds(0, 64)], histc_l)
        quals = jnp.int32(0)
        above_acc = jnp.int32(0)
        suf_at = jnp.int32(0)
        hc_at = jnp.int32(0)
        totals = []
        vregs = []
        for i in range(4):
            v = histc_l[pl.ds(i * 16, 16)]
            vregs.append(v)
            totals.append(jnp.sum(v))
        for i in range(4):
            above = sum([totals[j] for j in range(i + 1, 4)], jnp.int32(0))
            suf = _suffix(vregs[i]) + above
            quals += jnp.sum((suf >= rem).astype(jnp.int32))
        c_star = quals - 1
        for i in range(4):
            above = sum([totals[j] for j in range(i + 1, 4)], jnp.int32(0))
            suf = _suffix(vregs[i]) + above
            suf_at += _pick(suf, c_star - i * 16)
            hc_at += _pick(vregs[i], c_star - i * 16)
        rem = rem - (suf_at - hc_at)

        pltpu.sync_copy(histf.at[pl.ds(c_star * 16, 16)], chunk_l)
        v = chunk_l[...]
        suf = _suffix(v)
        fq = jnp.sum((suf >= rem).astype(jnp.int32))
        b_in = fq - 1
        rem = rem - (_pick(suf, b_in) - _pick(v, b_in))
        prefix = (prefix << 10) | (c_star * 16 + b_in)

    for g in range(NG):
        b = bitsv[pl.ds(g * 16, 16)]
        yg = yv[pl.ds(g * 16, 16)]
        ag = actv[pl.ds(g * 16, 16)]
        yk = jnp.where(b >= prefix, yg, 0.0)
        outv[pl.ds(g * 16, 16)] = ag + TAU * (yk - ag)
    pltpu.sync_copy(outv, out_hbm.at[pl.ds(base, CHUNK)])


@functools.partial(
    pl.kernel,
    out_type=jax.ShapeDtypeStruct((N_OUT,), jnp.float32),
    mesh=plsc.VectorSubcoreMesh(core_axis_name="c", subcore_axis_name="s",
                                num_cores=1),
    compiler_params=pltpu.CompilerParams(needs_layout_passes=False),
    scratch_types=[
        pltpu.VMEM((CHUNK,), jnp.float32),
        pltpu.VMEM((CHUNK,), jnp.float32),
        pltpu.VMEM((CHUNK,), jnp.float32),
        pltpu.VMEM((CHUNK,), jnp.int32),
        pltpu.VMEM((8, 128), jnp.int32),
        pltpu.VMEM((8, 128), jnp.int32),
        pltpu.VMEM((128,), jnp.int32),
        pltpu.VMEM((80,), jnp.int32),
        pltpu.VMEM((64,), jnp.int32),
        pltpu.VMEM((16,), jnp.int32),
        pltpu.SemaphoreType.DMA,
        pltpu.VMEM_SHARED((FINE_PAD,), jnp.int32),
        pltpu.VMEM_SHARED((FINE_PAD,), jnp.int32),
        pltpu.VMEM_SHARED((FINE_PAD,), jnp.int32),
        pltpu.VMEM_SHARED((COARSE_PAD,), jnp.int32),
        pltpu.VMEM_SHARED((COARSE_PAD,), jnp.int32),
        pltpu.VMEM_SHARED((COARSE_PAD,), jnp.int32),
    ],
)
def _sc_kwta(y_hbm, act_hbm, out_hbm, *scratch):
    _sc_body(y_hbm, act_hbm, out_hbm, *scratch)


@jax.jit
def kernel(a_ECin, activity, W):
    y = _matvec(a_ECin, W)
    return _sc_kwta(y, activity)


# speedup vs baseline: 1.6442x; 1.6442x over previous
import functools

import jax
import jax.numpy as jnp
from jax.experimental import pallas as pl
from jax.experimental.pallas import tpu as pltpu

N_IN = 4096
N_OUT = 16384
KTOP = max(1, int(0.01 * N_OUT))
TAU = 0.1
BC = 1024
NB = N_OUT // BC


def _body(a_ref, w_ref, act_ref, out_ref, y_ref):
    i = pl.program_id(0)
    x = jnp.maximum(
        jnp.dot(a_ref[...], w_ref[...], preferred_element_type=jnp.float32), 0.0)
    y_ref[:, pl.ds(i * BC, BC)] = x / (x + 1.0)

    @pl.when(i == NB - 1)
    def _epilogue():
        y = y_ref[...]
        bits = jax.lax.bitcast_convert_type(y, jnp.int32)

        widths = []
        w = 1 << 30
        while w > 1:
            w = (w + 2) // 3
            widths.append(w)

        lo = jnp.zeros((1, 128), jnp.int32)
        for w1 in widths:
            m1 = lo + w1
            m2 = lo + 2 * w1
            m1b = jnp.broadcast_to(m1[:, :1], (1, N_OUT))
            m2b = jnp.broadcast_to(m2[:, :1], (1, N_OUT))
            q = (jnp.where(bits >= m1b, 1, 0)
                 + jnp.where(bits >= m2b, 1 << 15, 0))
            tot = jnp.broadcast_to(
                jnp.sum(q, keepdims=True), (1, 128))
            c1 = tot & 0x7FFF
            c2 = tot >> 15
            lo = jnp.where(c2 >= KTOP, m2, jnp.where(c1 >= KTOP, m1, lo))

        thr = jnp.broadcast_to(lo[:, :1], (1, N_OUT))
        y_kwta = jnp.where(bits >= thr, y, 0.0)
        act = act_ref[...]
        out_ref[...] = act + TAU * (y_kwta - act)


@jax.jit
def kernel(a_ECin, activity, W):
    out = pl.pallas_call(
        _body,
        grid=(NB,),
        in_specs=[
            pl.BlockSpec((1, N_IN), lambda i: (0, 0)),
            pl.BlockSpec((N_IN, BC), lambda i: (0, i)),
            pl.BlockSpec((1, N_OUT), lambda i: (0, 0)),
        ],
        out_specs=pl.BlockSpec((1, N_OUT), lambda i: (0, 0)),
        out_shape=jax.ShapeDtypeStruct((1, N_OUT), jnp.float32),
        scratch_shapes=[pltpu.VMEM((1, N_OUT), jnp.float32)],
        compiler_params=pltpu.CompilerParams(
            dimension_semantics=("arbitrary",)),
    )(a_ECin.reshape(1, N_IN), W, activity.reshape(1, N_OUT))
    return out.reshape(N_OUT)


# speedup vs baseline: 1.6735x; 1.0178x over previous
import functools

import jax
import jax.numpy as jnp
from jax.experimental import pallas as pl
from jax.experimental.pallas import tpu as pltpu

N_IN = 4096
N_OUT = 16384
KTOP = max(1, int(0.01 * N_OUT))
TAU = 0.1
BC = 1024
NB = N_OUT // BC


NLAD = 64


def _body(a_ref, w_ref, act_ref, out_ref, y_ref, acc_ref):
    i = pl.program_id(0)

    @pl.when(i == 0)
    def _init():
        acc_ref[...] = jnp.zeros((NLAD, BC), jnp.float32)

    x = jnp.maximum(
        jnp.dot(a_ref[...], w_ref[...], preferred_element_type=jnp.float32), 0.0)
    yb = x / (x + 1.0)
    y_ref[:, pl.ds(i * BC, BC)] = yb

    bb = jax.lax.bitcast_convert_type(yb, jnp.int32)
    for t in range(NLAD):
        acc_ref[pl.ds(t, 1), :] += jnp.where(bb >= (t << 24), 1.0, 0.0)

    @pl.when(i == NB - 1)
    def _epilogue():
        y = y_ref[...]
        bits = jax.lax.bitcast_convert_type(y, jnp.int32)

        totals = jnp.dot(acc_ref[...], jnp.ones((BC, 1), jnp.float32),
                         preferred_element_type=jnp.float32)
        t_star = jnp.sum((totals >= KTOP).astype(jnp.int32)) - 1

        widths = []
        w = 1 << 24
        while w > 1:
            w = (w + 2) // 3
            widths.append(w)

        lo = jnp.zeros((1, 128), jnp.int32) + (t_star << 24)
        for w1 in widths:
            m1 = lo + w1
            m2 = lo + 2 * w1
            m1b = jnp.broadcast_to(m1[:, :1], (1, N_OUT))
            m2b = jnp.broadcast_to(m2[:, :1], (1, N_OUT))
            q = (jnp.where(bits >= m1b, 1, 0)
                 + jnp.where(bits >= m2b, 1 << 15, 0))
            tot = jnp.broadcast_to(
                jnp.sum(q, keepdims=True), (1, 128))
            c1 = tot & 0x7FFF
            c2 = tot >> 15
            lo = jnp.where(c2 >= KTOP, m2, jnp.where(c1 >= KTOP, m1, lo))

        thr = jnp.broadcast_to(lo[:, :1], (1, N_OUT))
        y_kwta = jnp.where(bits >= thr, y, 0.0)
        act = act_ref[...]
        out_ref[...] = act + TAU * (y_kwta - act)


@jax.jit
def kernel(a_ECin, activity, W):
    out = pl.pallas_call(
        _body,
        grid=(NB,),
        in_specs=[
            pl.BlockSpec((1, N_IN), lambda i: (0, 0)),
            pl.BlockSpec((N_IN, BC), lambda i: (0, i)),
            pl.BlockSpec((1, N_OUT), lambda i: (0, 0)),
        ],
        out_specs=pl.BlockSpec((1, N_OUT), lambda i: (0, 0)),
        out_shape=jax.ShapeDtypeStruct((1, N_OUT), jnp.float32),
        scratch_shapes=[pltpu.VMEM((1, N_OUT), jnp.float32),
                        pltpu.VMEM((NLAD, BC), jnp.float32)],
        compiler_params=pltpu.CompilerParams(
            dimension_semantics=("arbitrary",)),
    )(a_ECin.reshape(1, N_IN), W, activity.reshape(1, N_OUT))
    return out.reshape(N_OUT)


# speedup vs baseline: 1.6750x; 1.0009x over previous
import functools

import jax
import jax.numpy as jnp
from jax.experimental import pallas as pl
from jax.experimental.pallas import tpu as pltpu

N_IN = 4096
N_OUT = 16384
KTOP = max(1, int(0.01 * N_OUT))
TAU = 0.1
BC = 1024
NB = N_OUT // BC


NLAD = 128
LSH = 23


def _body(a_ref, w_ref, act_ref, out_ref, y_ref, acc_ref):
    i = pl.program_id(0)

    @pl.when(i == 0)
    def _init():
        acc_ref[...] = jnp.zeros((NLAD, BC), jnp.float32)

    x = jnp.maximum(
        jnp.dot(a_ref[...], w_ref[...], preferred_element_type=jnp.float32), 0.0)
    yb = x / (x + 1.0)
    y_ref[:, pl.ds(i * BC, BC)] = yb

    bb = jax.lax.bitcast_convert_type(yb, jnp.int32)
    for t in range(NLAD):
        acc_ref[pl.ds(t, 1), :] += jnp.where(bb >= (t << LSH), 1.0, 0.0)

    @pl.when(i == NB - 1)
    def _epilogue():
        y = y_ref[...]
        bits = jax.lax.bitcast_convert_type(y, jnp.int32)

        totals = jnp.dot(acc_ref[...], jnp.ones((BC, 1), jnp.float32),
                         preferred_element_type=jnp.float32)
        t_star = jnp.sum((totals >= KTOP).astype(jnp.int32)) - 1

        widths = []
        w = 1 << LSH
        while w > 1:
            w = (w + 2) // 3
            widths.append(w)

        lo = jnp.zeros((1, 128), jnp.int32) + (t_star << LSH)
        for w1 in widths:
            m1 = lo + w1
            m2 = lo + 2 * w1
            m1b = jnp.broadcast_to(m1[:, :1], (1, N_OUT))
            m2b = jnp.broadcast_to(m2[:, :1], (1, N_OUT))
            q = (jnp.where(bits >= m1b, 1, 0)
                 + jnp.where(bits >= m2b, 1 << 15, 0))
            tot = jnp.broadcast_to(
                jnp.sum(q, keepdims=True), (1, 128))
            c1 = tot & 0x7FFF
            c2 = tot >> 15
            lo = jnp.where(c2 >= KTOP, m2, jnp.where(c1 >= KTOP, m1, lo))

        thr = jnp.broadcast_to(lo[:, :1], (1, N_OUT))
        y_kwta = jnp.where(bits >= thr, y, 0.0)
        act = act_ref[...]
        out_ref[...] = act + TAU * (y_kwta - act)


@jax.jit
def kernel(a_ECin, activity, W):
    out = pl.pallas_call(
        _body,
        grid=(NB,),
        in_specs=[
            pl.BlockSpec((1, N_IN), lambda i: (0, 0)),
            pl.BlockSpec((N_IN, BC), lambda i: (0, i)),
            pl.BlockSpec((1, N_OUT), lambda i: (0, 0)),
        ],
        out_specs=pl.BlockSpec((1, N_OUT), lambda i: (0, 0)),
        out_shape=jax.ShapeDtypeStruct((1, N_OUT), jnp.float32),
        scratch_shapes=[pltpu.VMEM((1, N_OUT), jnp.float32),
                        pltpu.VMEM((NLAD, BC), jnp.float32)],
        compiler_params=pltpu.CompilerParams(
            dimension_semantics=("arbitrary",)),
    )(a_ECin.reshape(1, N_IN), W, activity.reshape(1, N_OUT))
    return out.reshape(N_OUT)
